# baseline (speedup 1.0000x reference)
import functools

import jax
import jax.numpy as jnp
from jax import lax
from jax.experimental import pallas as pl
from jax.experimental.pallas import tpu as pltpu
from jax.experimental.pallas import tpu_sc as plsc

NC = 2
NS = 16
NW = NC * NS
C = 128
NBUF = 2

_mesh = plsc.VectorSubcoreMesh(
    core_axis_name="c", subcore_axis_name="s", num_cores=NC, num_subcores=NS
)
_sc_params = pltpu.CompilerParams(
    needs_layout_passes=False, use_tc_tiling_on_sc=False
)


def _make_deg_kernel(n_pad, ch):
    @functools.partial(
        pl.kernel,
        out_type=jax.ShapeDtypeStruct((NW, n_pad), jnp.float32),
        mesh=_mesh,
        compiler_params=_sc_params,
        scratch_types=[
            pltpu.VMEM((n_pad,), jnp.float32),
            pltpu.VMEM((ch, C), jnp.int32),
        ],
    )
    def deg_kernel(dst_hbm, out_hbm, hist, didx):
        c = lax.axis_index("c")
        s = lax.axis_index("s")
        wid = s * NC + c

        def zero_body(i, _):
            hist[pl.ds(i * 16, 16)] = jnp.zeros((16,), jnp.float32)
            return 0

        lax.fori_loop(0, n_pad // 16, zero_body, 0)
        pltpu.sync_copy(dst_hbm.at[wid], didx)

        ones = jnp.full((16,), 1.0, jnp.float32)

        def body(k, _):
            for j in range(C // 16):
                idx16 = didx[k, pl.ds(j * 16, 16)]
                plsc.addupdate_scatter(hist, [idx16], ones)
            return 0

        lax.fori_loop(0, ch, body, 0)
        pltpu.sync_copy(hist, out_hbm.at[wid])

    return deg_kernel


def _make_prop_kernel(n, n_pad, ch, d, rows_per):
    groups = ch // NBUF

    @functools.partial(
        pl.kernel,
        out_type=jax.ShapeDtypeStruct((NC, n, d), jnp.float32),
        mesh=_mesh,
        compiler_params=_sc_params,
        scratch_types=[
            pltpu.VMEM_SHARED((n_pad, d), jnp.float32),
            pltpu.VMEM((ch, C), jnp.int32),
            pltpu.VMEM((NBUF, C), jnp.int32),
            tuple(pltpu.VMEM((C, d), jnp.float32) for _ in range(NBUF)),
            pltpu.SemaphoreType.DMA((NBUF,)),
            pltpu.SemaphoreType.DMA((NBUF,)),
            pltpu.SemaphoreType.DMA((NBUF,)),
        ],
    )
    def prop_kernel(p_hbm, src_hbm, dst_hbm, out_hbm, acc, sidx, didx, rows,
                    gsem, ssem, isem):
        c = lax.axis_index("c")
        s = lax.axis_index("s")
        wid = s * NC + c

        tail = n - NS * rows_per
        pltpu.sync_copy(
            p_hbm.at[pl.ds(s * rows_per, rows_per)],
            acc.at[pl.ds(s * rows_per, rows_per)],
        )
        @pl.when(s == 0)
        def _():
            pltpu.sync_copy(
                p_hbm.at[pl.ds(NS * rows_per, tail)],
                acc.at[pl.ds(NS * rows_per, tail)],
            )
        pltpu.sync_copy(src_hbm.at[wid], sidx)

        def fire_gather(k, b):
            pltpu.async_copy(p_hbm.at[sidx.at[k]], rows[b], gsem.at[b])

        def wait_gather(k, b):
            pltpu.make_async_copy(p_hbm.at[sidx.at[k]], rows[b], gsem.at[b]).wait()

        def fire_scatter(k, b):
            pltpu.async_copy(rows[b], acc.at[didx.at[b]], ssem.at[b], add=True)

        def wait_scatter(k, b):
            pltpu.make_async_copy(rows[b], acc.at[didx.at[b]], ssem.at[b]).wait()

        def fire_didx(k, b):
            pltpu.async_copy(dst_hbm.at[wid, k], didx.at[b], isem.at[b])

        def wait_didx(k, b):
            pltpu.make_async_copy(dst_hbm.at[wid, k], didx.at[b], isem.at[b]).wait()

        for b in range(NBUF):
            fire_didx(b, b)
            fire_gather(b, b)
        plsc.subcore_barrier()

        def body(g, _):
            base = g * NBUF
            for b in range(NBUF):
                wait_gather(base + b, b)
                wait_didx(base + b, b)
                fire_scatter(base + b, b)
            for b in range(NBUF):
                wait_scatter(base + b, b)

                @pl.when(g + 1 < groups)
                def _():
                    fire_didx(base + NBUF + b, b)
                    fire_gather(base + NBUF + b, b)
            return 0

        lax.fori_loop(0, groups, body, 0)
        plsc.subcore_barrier()
        pltpu.sync_copy(
            acc.at[pl.ds(s * rows_per, rows_per)],
            out_hbm.at[c, pl.ds(s * rows_per, rows_per)],
        )
        @pl.when(s == 0)
        def _():
            pltpu.sync_copy(
                acc.at[pl.ds(NS * rows_per, tail)],
                out_hbm.at[c, pl.ds(NS * rows_per, tail)],
            )

    return prop_kernel


def _deg_reduce_body(hist_ref, dis_ref):
    deg = 1.0 + jnp.sum(hist_ref[...], axis=0, keepdims=True)
    dis_ref[...] = lax.rsqrt(deg)


def _t0_body(x_ref, dis_ref, w_ref, p_ref):
    q = jnp.dot(x_ref[...], w_ref[...], preferred_element_type=jnp.float32)
    p_ref[...] = q * dis_ref[...]


def _t_mid_body(acc_ref, p_ref, dis_ref, b_ref, w_ref, o_ref):
    dis = dis_ref[...]
    a = acc_ref[...]
    h = (a[0] + a[1] - p_ref[...]) * dis + b_ref[...]
    h = jnp.maximum(h, 0.0)
    o_ref[...] = jnp.dot(h, w_ref[...], preferred_element_type=jnp.float32) * dis


def _t_final_body(acc_ref, p_ref, dis_ref, b_ref, o_ref):
    a = acc_ref[...]
    o_ref[...] = (a[0] + a[1] - p_ref[...]) * dis_ref[...] + b_ref[...]


def kernel(x, edge_index, batch, W1, b1, W2, b2, W3, b3):
    n, d_in = x.shape
    e = edge_index.shape[1]
    d_hid = W1.shape[1]
    n_cls = W3.shape[1]

    slab = NW * C * NBUF
    e_pad = -(-e // slab) * slab
    ch = e_pad // (NW * C)
    n_pad = -(-(n + 1) // NS) * NS
    rows_per = (n // NS) // 8 * 8

    src = edge_index[0]
    dst = edge_index[1]
    pad = e_pad - e
    src_p = jnp.concatenate([src, jnp.zeros((pad,), jnp.int32)]).reshape(NW, ch, C)
    dst_p = jnp.concatenate([dst, jnp.full((pad,), n, jnp.int32)]).reshape(NW, ch, C)

    hist = _make_deg_kernel(n_pad, ch)(dst_p)

    bn = 400
    grid = n // bn
    f32 = jnp.float32

    dis_full = pl.pallas_call(
        _deg_reduce_body,
        grid=(1,),
        in_specs=[pl.BlockSpec((NW, n_pad), lambda j: (0, 0))],
        out_specs=pl.BlockSpec((1, n_pad), lambda j: (0, 0)),
        out_shape=jax.ShapeDtypeStruct((1, n_pad), f32),
    )(hist)
    dis = dis_full[0, :n].reshape(n, 1)

    p1 = pl.pallas_call(
        _t0_body,
        grid=(grid,),
        in_specs=[
            pl.BlockSpec((bn, d_in), lambda j: (j, 0)),
            pl.BlockSpec((bn, 1), lambda j: (j, 0)),
            pl.BlockSpec((d_in, d_hid), lambda j: (0, 0)),
        ],
        out_specs=pl.BlockSpec((bn, d_hid), lambda j: (j, 0)),
        out_shape=jax.ShapeDtypeStruct((n, d_hid), f32),
    )(x, dis, W1)

    prop_h = _make_prop_kernel(n, n_pad, ch, d_hid, rows_per)
    prop_c = _make_prop_kernel(n, n_pad, ch, n_cls, rows_per)

    def t_mid(acc, p, dis, b, w, d_out):
        return pl.pallas_call(
            _t_mid_body,
            grid=(grid,),
            in_specs=[
                pl.BlockSpec((NC, bn, d_hid), lambda j: (0, j, 0)),
                pl.BlockSpec((bn, d_hid), lambda j: (j, 0)),
                pl.BlockSpec((bn, 1), lambda j: (j, 0)),
                pl.BlockSpec((1, d_hid), lambda j: (0, 0)),
                pl.BlockSpec((d_hid, d_out), lambda j: (0, 0)),
            ],
            out_specs=pl.BlockSpec((bn, d_out), lambda j: (j, 0)),
            out_shape=jax.ShapeDtypeStruct((n, d_out), f32),
        )(acc, p, dis, b.reshape(1, -1), w)

    a1 = prop_h(p1, src_p, dst_p)
    p2 = t_mid(a1, p1, dis, b1, W2, d_hid)
    a2 = prop_h(p2, src_p, dst_p)
    p3 = t_mid(a2, p2, dis, b2, W3, n_cls)
    a3 = prop_c(p3, src_p, dst_p)

    out = pl.pallas_call(
        _t_final_body,
        grid=(grid,),
        in_specs=[
            pl.BlockSpec((NC, bn, n_cls), lambda j: (0, j, 0)),
            pl.BlockSpec((bn, n_cls), lambda j: (j, 0)),
            pl.BlockSpec((bn, 1), lambda j: (j, 0)),
            pl.BlockSpec((1, n_cls), lambda j: (0, 0)),
        ],
        out_specs=pl.BlockSpec((bn, n_cls), lambda j: (j, 0)),
        out_shape=jax.ShapeDtypeStruct((n, n_cls), f32),
    )(a3, p3, dis, b3.reshape(1, -1))
    return out


# speedup vs baseline: 1.7698x; 1.7698x over previous
import functools

import jax
import jax.numpy as jnp
from jax import lax
from jax.experimental import pallas as pl
from jax.experimental.pallas import tpu as pltpu
from jax.experimental.pallas import tpu_sc as plsc

NC = 2
NS = 16
NW = NC * NS
C = 112
NBUF = 2

_mesh = plsc.VectorSubcoreMesh(
    core_axis_name="c", subcore_axis_name="s", num_cores=NC, num_subcores=NS
)
_sc_params = pltpu.CompilerParams(
    needs_layout_passes=False, use_tc_tiling_on_sc=False
)


def _make_deg_kernel(n_pad, ch):
    @functools.partial(
        pl.kernel,
        out_type=jax.ShapeDtypeStruct((NW, n_pad), jnp.float32),
        mesh=_mesh,
        compiler_params=_sc_params,
        scratch_types=[
            pltpu.VMEM((n_pad,), jnp.float32),
            pltpu.VMEM((ch, C), jnp.int32),
        ],
    )
    def deg_kernel(dst_hbm, out_hbm, hist, didx):
        c = lax.axis_index("c")
        s = lax.axis_index("s")
        wid = s * NC + c

        def zero_body(i, _):
            hist[pl.ds(i * 16, 16)] = jnp.zeros((16,), jnp.float32)
            return 0

        lax.fori_loop(0, n_pad // 16, zero_body, 0)
        pltpu.sync_copy(dst_hbm.at[wid], didx)

        ones = jnp.full((16,), 1.0, jnp.float32)

        def body(k, _):
            for j in range(C // 16):
                idx16 = didx[k, pl.ds(j * 16, 16)]
                plsc.addupdate_scatter(hist, [idx16], ones)
            return 0

        lax.fori_loop(0, ch, body, 0)
        pltpu.sync_copy(hist, out_hbm.at[wid])

    return deg_kernel


def _make_prop_kernel(n, n_pad, ch, d, rows_per):
    groups = ch // NBUF

    @functools.partial(
        pl.kernel,
        out_type=jax.ShapeDtypeStruct((NC, n, d), jnp.float32),
        mesh=_mesh,
        compiler_params=_sc_params,
        scratch_types=[
            pltpu.VMEM_SHARED((n_pad, d), jnp.float32),
            pltpu.VMEM((ch, C), jnp.int32),
            pltpu.VMEM((ch, C), jnp.int32),
            tuple(pltpu.VMEM((C, d), jnp.float32) for _ in range(NBUF)),
            pltpu.SemaphoreType.DMA((NBUF,)),
        ],
    )
    def prop_kernel(p_hbm, src_hbm, dst_hbm, out_hbm, acc, sidx, didx, rows,
                    gsem):
        c = lax.axis_index("c")
        s = lax.axis_index("s")
        wid = s * NC + c

        tail = n - NS * rows_per
        pltpu.sync_copy(
            p_hbm.at[pl.ds(s * rows_per, rows_per)],
            acc.at[pl.ds(s * rows_per, rows_per)],
        )
        @pl.when(s == 0)
        def _():
            pltpu.sync_copy(
                p_hbm.at[pl.ds(NS * rows_per, tail)],
                acc.at[pl.ds(NS * rows_per, tail)],
            )
        pltpu.sync_copy(src_hbm.at[wid], sidx)
        pltpu.sync_copy(dst_hbm.at[wid], didx)

        def fire_gather(k, b):
            pltpu.async_copy(p_hbm.at[sidx.at[k]], rows[b], gsem.at[b])

        def wait_gather(k, b):
            pltpu.make_async_copy(p_hbm.at[sidx.at[k]], rows[b], gsem.at[b]).wait()

        fire_gather(0, 0)
        plsc.subcore_barrier()

        def body(g, _):
            for b in range(NBUF):
                k = g * NBUF + b
                wait_gather(k, b)

                @pl.when(k + 1 < ch)
                def _():
                    fire_gather(k + 1, 1 - b)

                pltpu.sync_copy(rows[b], acc.at[didx.at[k]], add=True)
            return 0

        lax.fori_loop(0, groups, body, 0)
        plsc.subcore_barrier()
        pltpu.sync_copy(
            acc.at[pl.ds(s * rows_per, rows_per)],
            out_hbm.at[c, pl.ds(s * rows_per, rows_per)],
        )
        @pl.when(s == 0)
        def _():
            pltpu.sync_copy(
                acc.at[pl.ds(NS * rows_per, tail)],
                out_hbm.at[c, pl.ds(NS * rows_per, tail)],
            )

    return prop_kernel


def _deg_reduce_body(hist_ref, dis_ref):
    deg = 1.0 + jnp.sum(hist_ref[...], axis=0, keepdims=True)
    dis_ref[...] = lax.rsqrt(deg)


def _t0_body(x_ref, dis_ref, w_ref, p_ref):
    q = jnp.dot(x_ref[...], w_ref[...], preferred_element_type=jnp.float32)
    p_ref[...] = q * dis_ref[...]


def _t_mid_body(acc_ref, p_ref, dis_ref, b_ref, w_ref, o_ref):
    dis = dis_ref[...]
    a = acc_ref[...]
    h = (a[0] + a[1] - p_ref[...]) * dis + b_ref[...]
    h = jnp.maximum(h, 0.0)
    o_ref[...] = jnp.dot(h, w_ref[...], preferred_element_type=jnp.float32) * dis


def _t_final_body(acc_ref, p_ref, dis_ref, b_ref, o_ref):
    a = acc_ref[...]
    o_ref[...] = (a[0] + a[1] - p_ref[...]) * dis_ref[...] + b_ref[...]


def kernel(x, edge_index, batch, W1, b1, W2, b2, W3, b3):
    n, d_in = x.shape
    e = edge_index.shape[1]
    d_hid = W1.shape[1]
    n_cls = W3.shape[1]

    slab = NW * C * NBUF
    e_pad = -(-e // slab) * slab
    ch = e_pad // (NW * C)
    n_pad = -(-(n + 1) // NS) * NS
    rows_per = (n // NS) // 8 * 8

    src = edge_index[0]
    dst = edge_index[1]
    pad = e_pad - e
    src_p = jnp.concatenate([src, jnp.zeros((pad,), jnp.int32)]).reshape(NW, ch, C)
    dst_p = jnp.concatenate([dst, jnp.full((pad,), n, jnp.int32)]).reshape(NW, ch, C)

    hist = _make_deg_kernel(n_pad, ch)(dst_p)

    bn = 400
    grid = n // bn
    f32 = jnp.float32

    dis_full = pl.pallas_call(
        _deg_reduce_body,
        grid=(1,),
        in_specs=[pl.BlockSpec((NW, n_pad), lambda j: (0, 0))],
        out_specs=pl.BlockSpec((1, n_pad), lambda j: (0, 0)),
        out_shape=jax.ShapeDtypeStruct((1, n_pad), f32),
    )(hist)
    dis = dis_full[0, :n].reshape(n, 1)

    p1 = pl.pallas_call(
        _t0_body,
        grid=(grid,),
        in_specs=[
            pl.BlockSpec((bn, d_in), lambda j: (j, 0)),
            pl.BlockSpec((bn, 1), lambda j: (j, 0)),
            pl.BlockSpec((d_in, d_hid), lambda j: (0, 0)),
        ],
        out_specs=pl.BlockSpec((bn, d_hid), lambda j: (j, 0)),
        out_shape=jax.ShapeDtypeStruct((n, d_hid), f32),
    )(x, dis, W1)

    prop_h = _make_prop_kernel(n, n_pad, ch, d_hid, rows_per)
    prop_c = _make_prop_kernel(n, n_pad, ch, n_cls, rows_per)

    def t_mid(acc, p, dis, b, w, d_out):
        return pl.pallas_call(
            _t_mid_body,
            grid=(grid,),
            in_specs=[
                pl.BlockSpec((NC, bn, d_hid), lambda j: (0, j, 0)),
                pl.BlockSpec((bn, d_hid), lambda j: (j, 0)),
                pl.BlockSpec((bn, 1), lambda j: (j, 0)),
                pl.BlockSpec((1, d_hid), lambda j: (0, 0)),
                pl.BlockSpec((d_hid, d_out), lambda j: (0, 0)),
            ],
            out_specs=pl.BlockSpec((bn, d_out), lambda j: (j, 0)),
            out_shape=jax.ShapeDtypeStruct((n, d_out), f32),
        )(acc, p, dis, b.reshape(1, -1), w)

    a1 = prop_h(p1, src_p, dst_p)
    p2 = t_mid(a1, p1, dis, b1, W2, d_hid)
    a2 = prop_h(p2, src_p, dst_p)
    p3 = t_mid(a2, p2, dis, b2, W3, n_cls)
    a3 = prop_c(p3, src_p, dst_p)

    out = pl.pallas_call(
        _t_final_body,
        grid=(grid,),
        in_specs=[
            pl.BlockSpec((NC, bn, n_cls), lambda j: (0, j, 0)),
            pl.BlockSpec((bn, n_cls), lambda j: (j, 0)),
            pl.BlockSpec((bn, 1), lambda j: (j, 0)),
            pl.BlockSpec((1, n_cls), lambda j: (0, 0)),
        ],
        out_specs=pl.BlockSpec((bn, n_cls), lambda j: (j, 0)),
        out_shape=jax.ShapeDtypeStruct((n, n_cls), f32),
    )(a3, p3, dis, b3.reshape(1, -1))
    return out


# speedup vs baseline: 2.0127x; 1.1373x over previous
import functools

import jax
import jax.numpy as jnp
from jax import lax
from jax.experimental import pallas as pl
from jax.experimental.pallas import tpu as pltpu
from jax.experimental.pallas import tpu_sc as plsc

NC = 2
NS = 16
NW = NC * NS
C = 80
NBUF = 3

_mesh = plsc.VectorSubcoreMesh(
    core_axis_name="c", subcore_axis_name="s", num_cores=NC, num_subcores=NS
)
_sc_params = pltpu.CompilerParams(
    needs_layout_passes=False, use_tc_tiling_on_sc=False
)


def _make_deg_kernel(n_pad, ch):
    @functools.partial(
        pl.kernel,
        out_type=jax.ShapeDtypeStruct((NW, n_pad), jnp.float32),
        mesh=_mesh,
        compiler_params=_sc_params,
        scratch_types=[
            pltpu.VMEM((n_pad,), jnp.float32),
            pltpu.VMEM((ch, C), jnp.int32),
        ],
    )
    def deg_kernel(dst_hbm, out_hbm, hist, didx):
        c = lax.axis_index("c")
        s = lax.axis_index("s")
        wid = s * NC + c

        def zero_body(i, _):
            hist[pl.ds(i * 16, 16)] = jnp.zeros((16,), jnp.float32)
            return 0

        lax.fori_loop(0, n_pad // 16, zero_body, 0)
        pltpu.sync_copy(dst_hbm.at[wid], didx)

        ones = jnp.full((16,), 1.0, jnp.float32)

        def body(k, _):
            for j in range(C // 16):
                idx16 = didx[k, pl.ds(j * 16, 16)]
                plsc.addupdate_scatter(hist, [idx16], ones)
            return 0

        lax.fori_loop(0, ch, body, 0)
        pltpu.sync_copy(hist, out_hbm.at[wid])

    return deg_kernel


def _make_prop_kernel(n, n_pad, ch, d, rows_per):
    groups = ch // NBUF

    @functools.partial(
        pl.kernel,
        out_type=jax.ShapeDtypeStruct((NC, n, d), jnp.float32),
        mesh=_mesh,
        compiler_params=_sc_params,
        scratch_types=[
            pltpu.VMEM_SHARED((n_pad, d), jnp.float32),
            pltpu.VMEM((ch, C), jnp.int32),
            pltpu.VMEM((ch, C), jnp.int32),
            tuple(pltpu.VMEM((C, d), jnp.float32) for _ in range(NBUF)),
            pltpu.SemaphoreType.DMA((NBUF,)),
        ],
    )
    def prop_kernel(p_hbm, src_hbm, dst_hbm, out_hbm, acc, sidx, didx, rows,
                    gsem):
        c = lax.axis_index("c")
        s = lax.axis_index("s")
        wid = s * NC + c

        tail = n - NS * rows_per
        pltpu.sync_copy(
            p_hbm.at[pl.ds(s * rows_per, rows_per)],
            acc.at[pl.ds(s * rows_per, rows_per)],
        )
        @pl.when(s == 0)
        def _():
            pltpu.sync_copy(
                p_hbm.at[pl.ds(NS * rows_per, tail)],
                acc.at[pl.ds(NS * rows_per, tail)],
            )
        pltpu.sync_copy(src_hbm.at[wid], sidx)
        pltpu.sync_copy(dst_hbm.at[wid], didx)

        def fire_gather(k, b):
            pltpu.async_copy(p_hbm.at[sidx.at[k]], rows[b], gsem.at[b])

        def wait_gather(k, b):
            pltpu.make_async_copy(p_hbm.at[sidx.at[k]], rows[b], gsem.at[b]).wait()

        fire_gather(0, 0)
        fire_gather(1, 1)
        plsc.subcore_barrier()

        def body(g, _):
            for b in range(NBUF):
                k = g * NBUF + b
                wait_gather(k, b)

                @pl.when(k + NBUF - 1 < ch)
                def _():
                    fire_gather(k + NBUF - 1, (b + NBUF - 1) % NBUF)

                pltpu.sync_copy(rows[b], acc.at[didx.at[k]], add=True)
            return 0

        lax.fori_loop(0, groups, body, 0)
        plsc.subcore_barrier()
        pltpu.sync_copy(
            acc.at[pl.ds(s * rows_per, rows_per)],
            out_hbm.at[c, pl.ds(s * rows_per, rows_per)],
        )
        @pl.when(s == 0)
        def _():
            pltpu.sync_copy(
                acc.at[pl.ds(NS * rows_per, tail)],
                out_hbm.at[c, pl.ds(NS * rows_per, tail)],
            )

    return prop_kernel


def _deg_reduce_body(hist_ref, dis_ref):
    deg = 1.0 + jnp.sum(hist_ref[...], axis=0, keepdims=True)
    dis_ref[...] = lax.rsqrt(deg)


def _t0_body(x_ref, dis_ref, w_ref, p_ref):
    q = jnp.dot(x_ref[...], w_ref[...], preferred_element_type=jnp.float32)
    p_ref[...] = q * dis_ref[...]


def _t_mid_body(acc_ref, p_ref, dis_ref, b_ref, w_ref, o_ref):
    dis = dis_ref[...]
    a = acc_ref[...]
    h = (a[0] + a[1] - p_ref[...]) * dis + b_ref[...]
    h = jnp.maximum(h, 0.0)
    o_ref[...] = jnp.dot(h, w_ref[...], preferred_element_type=jnp.float32) * dis


def _t_final_body(acc_ref, p_ref, dis_ref, b_ref, o_ref):
    a = acc_ref[...]
    o_ref[...] = (a[0] + a[1] - p_ref[...]) * dis_ref[...] + b_ref[...]


def kernel(x, edge_index, batch, W1, b1, W2, b2, W3, b3):
    n, d_in = x.shape
    e = edge_index.shape[1]
    d_hid = W1.shape[1]
    n_cls = W3.shape[1]

    slab = NW * C * NBUF
    e_pad = -(-e // slab) * slab
    ch = e_pad // (NW * C)
    n_pad = -(-(n + 1) // NS) * NS
    rows_per = (n // NS) // 8 * 8

    src = edge_index[0]
    dst = edge_index[1]
    pad = e_pad - e
    src_p = jnp.concatenate([src, jnp.zeros((pad,), jnp.int32)]).reshape(NW, ch, C)
    dst_p = jnp.concatenate([dst, jnp.full((pad,), n, jnp.int32)]).reshape(NW, ch, C)

    hist = _make_deg_kernel(n_pad, ch)(dst_p)

    bn = 400
    grid = n // bn
    f32 = jnp.float32

    dis_full = pl.pallas_call(
        _deg_reduce_body,
        grid=(1,),
        in_specs=[pl.BlockSpec((NW, n_pad), lambda j: (0, 0))],
        out_specs=pl.BlockSpec((1, n_pad), lambda j: (0, 0)),
        out_shape=jax.ShapeDtypeStruct((1, n_pad), f32),
    )(hist)
    dis = dis_full[0, :n].reshape(n, 1)

    p1 = pl.pallas_call(
        _t0_body,
        grid=(grid,),
        in_specs=[
            pl.BlockSpec((bn, d_in), lambda j: (j, 0)),
            pl.BlockSpec((bn, 1), lambda j: (j, 0)),
            pl.BlockSpec((d_in, d_hid), lambda j: (0, 0)),
        ],
        out_specs=pl.BlockSpec((bn, d_hid), lambda j: (j, 0)),
        out_shape=jax.ShapeDtypeStruct((n, d_hid), f32),
    )(x, dis, W1)

    prop_h = _make_prop_kernel(n, n_pad, ch, d_hid, rows_per)
    prop_c = _make_prop_kernel(n, n_pad, ch, n_cls, rows_per)

    def t_mid(acc, p, dis, b, w, d_out):
        return pl.pallas_call(
            _t_mid_body,
            grid=(grid,),
            in_specs=[
                pl.BlockSpec((NC, bn, d_hid), lambda j: (0, j, 0)),
                pl.BlockSpec((bn, d_hid), lambda j: (j, 0)),
                pl.BlockSpec((bn, 1), lambda j: (j, 0)),
                pl.BlockSpec((1, d_hid), lambda j: (0, 0)),
                pl.BlockSpec((d_hid, d_out), lambda j: (0, 0)),
            ],
            out_specs=pl.BlockSpec((bn, d_out), lambda j: (j, 0)),
            out_shape=jax.ShapeDtypeStruct((n, d_out), f32),
        )(acc, p, dis, b.reshape(1, -1), w)

    a1 = prop_h(p1, src_p, dst_p)
    p2 = t_mid(a1, p1, dis, b1, W2, d_hid)
    a2 = prop_h(p2, src_p, dst_p)
    p3 = t_mid(a2, p2, dis, b2, W3, n_cls)
    a3 = prop_c(p3, src_p, dst_p)

    out = pl.pallas_call(
        _t_final_body,
        grid=(grid,),
        in_specs=[
            pl.BlockSpec((NC, bn, n_cls), lambda j: (0, j, 0)),
            pl.BlockSpec((bn, n_cls), lambda j: (j, 0)),
            pl.BlockSpec((bn, 1), lambda j: (j, 0)),
            pl.BlockSpec((1, n_cls), lambda j: (0, 0)),
        ],
        out_specs=pl.BlockSpec((bn, n_cls), lambda j: (j, 0)),
        out_shape=jax.ShapeDtypeStruct((n, n_cls), f32),
    )(a3, p3, dis, b3.reshape(1, -1))
    return out
